# Initial kernel scaffold; baseline (speedup 1.0000x reference)
import jax
import jax.numpy as jnp
from jax.experimental import pallas as pl


def kernel(inputs, tag_table):
    raise NotImplementedError("write your pallas kernel here")


# speedup vs baseline: 1.0529x; 1.0529x over previous
import functools

import jax
import jax.numpy as jnp
from jax import lax
from jax.experimental import pallas as pl
from jax.experimental.pallas import tpu as pltpu
from jax.experimental.pallas import tpu_sc as plsc

VOCAB = 50
BATCH = 16384
SEQ = 200

NW = 32
RPW = BATCH // NW
GROUP = 16
NGROUPS = RPW // GROUP
TBL = 64
NFULL = SEQ // 16
TAIL = NFULL * 16 - (16 - (SEQ - NFULL * 16))
NDUP = 16 - (SEQ - NFULL * 16)


def _build():
    mesh = plsc.VectorSubcoreMesh(core_axis_name="c", subcore_axis_name="s")

    @functools.partial(
        pl.kernel,
        mesh=mesh,
        out_type=jax.ShapeDtypeStruct((BATCH,), jnp.float32),
        compiler_params=pltpu.CompilerParams(needs_layout_passes=False),
        scratch_types=[
            pltpu.VMEM((RPW * SEQ,), jnp.int32),
            pltpu.VMEM((TBL,), jnp.float32),
            pltpu.VMEM((GROUP * 16,), jnp.float32),
            pltpu.VMEM((RPW,), jnp.float32),
        ],
    )
    def k(idx_hbm, table_hbm, out_hbm, idx_v, table_v, red_v, out_v):
        wid = lax.axis_index("s") * 2 + lax.axis_index("c")
        pltpu.sync_copy(table_hbm, table_v)
        pltpu.sync_copy(idx_hbm.at[pl.ds(wid * (RPW * SEQ), RPW * SEQ)], idx_v)

        lane = lax.iota(jnp.int32, 16)
        tail_dup = lane < NDUP
        colbase = lane * 16

        def group_body(g, carry):
            goff = g * (GROUP * SEQ)
            for r in range(GROUP):
                roff = goff + r * SEQ
                acc = plsc.load_gather(table_v, [idx_v[pl.ds(roff, 16)]])
                for j in range(1, NFULL):
                    iv = idx_v[pl.ds(roff + j * 16, 16)]
                    acc = acc + plsc.load_gather(table_v, [iv])
                iv = idx_v[pl.ds(roff + TAIL, 16)]
                iv = jnp.where(tail_dup, TBL - 1, iv)
                acc = acc + plsc.load_gather(table_v, [iv])
                red_v[pl.ds(r * 16, 16)] = acc
            rowsum = plsc.load_gather(red_v, [colbase])
            for col in range(1, 16):
                rowsum = rowsum + plsc.load_gather(red_v, [colbase + col])
            out_v[pl.ds(g * GROUP, GROUP)] = rowsum
            return carry

        lax.fori_loop(0, NGROUPS, group_body, 0)
        pltpu.sync_copy(out_v, out_hbm.at[pl.ds(wid * RPW, RPW)])

    return k


_sc_kernel = _build()


@jax.jit
def kernel(inputs, tag_table):
    idx_flat = inputs.reshape(-1)
    table_pad = jnp.concatenate(
        [tag_table, jnp.zeros((TBL - VOCAB,), jnp.float32)]
    )
    return _sc_kernel(idx_flat, table_pad)
